# baseline (speedup 1.0000x reference)
import functools

import jax
import jax.numpy as jnp
from jax import lax
from jax.experimental import pallas as pl
from jax.experimental.pallas import tpu as pltpu
from jax.experimental.pallas import tpu_sc as plsc

BATCH = 16384
HIST = 50
HIST_PAD = 128
DIM = 64

NC = 2
NS = 16
NW = NC * NS
BPW = BATCH // NW
NB = 8
NGROUPS = BPW // NB
NPAIRS = NGROUPS // 2


def _gather_kernel(idx_hbm, table_hbm, out_hbm, idx_all, rows_v, gsem, ssem):
    wid = lax.axis_index("s") * NC + lax.axis_index("c")
    b0 = wid * BPW
    pltpu.sync_copy(idx_hbm.at[pl.ds(b0 * HIST_PAD, BPW * HIST_PAD)], idx_all)

    def fire_gathers(g, b):
        for i in range(NB):
            pltpu.async_copy(
                table_hbm.at[idx_all.at[pl.ds((g * NB + i) * HIST_PAD, HIST)]],
                rows_v.at[b, i],
                gsem,
            )

    def wait_gathers(b):
        for i in range(NB):
            pltpu.make_async_copy(
                table_hbm.at[idx_all.at[pl.ds(i * HIST_PAD, HIST)]],
                rows_v.at[b, i],
                gsem,
            ).wait()

    def start_store(g, b):
        pltpu.async_copy(rows_v.at[b], out_hbm.at[pl.ds(b0 + g * NB, NB)], ssem)

    def wait_store():
        pltpu.make_async_copy(
            rows_v.at[0], out_hbm.at[pl.ds(b0, NB)], ssem
        ).wait()

    fire_gathers(0, 0)

    def body(p, carry):
        g0 = p * 2
        wait_gathers(0)
        start_store(g0, 0)

        @pl.when(p > 0)
        def _():
            wait_store()

        fire_gathers(g0 + 1, 1)
        wait_gathers(1)
        start_store(g0 + 1, 1)
        wait_store()

        @pl.when(p < NPAIRS - 1)
        def _():
            fire_gathers(g0 + 2, 0)

        return carry

    lax.fori_loop(0, NPAIRS, body, 0)
    wait_store()


def kernel(inp, table):
    idx_pad = jnp.pad(inp.astype(jnp.int32), ((0, 0), (0, HIST_PAD - HIST)))
    idx_flat = idx_pad.reshape(BATCH * HIST_PAD)
    mesh = plsc.VectorSubcoreMesh(core_axis_name="c", subcore_axis_name="s")
    run = functools.partial(
        pl.kernel,
        mesh=mesh,
        out_type=jax.ShapeDtypeStruct((BATCH, HIST, DIM), jnp.float32),
        scratch_types=[
            pltpu.VMEM((BPW * HIST_PAD,), jnp.int32),
            pltpu.VMEM((2, NB, HIST, DIM), jnp.float32),
            pltpu.SemaphoreType.DMA,
            pltpu.SemaphoreType.DMA,
        ],
        compiler_params=pltpu.CompilerParams(use_tc_tiling_on_sc=False),
    )(_gather_kernel)
    return run(idx_flat, table)
